# Initial kernel scaffold; baseline (speedup 1.0000x reference)
import jax
import jax.numpy as jnp
from jax.experimental import pallas as pl


def kernel(src_node_features, neighbor_node_features, W_agg, W_self):
    raise NotImplementedError("write your pallas kernel here")


# speedup vs baseline: 1.3915x; 1.3915x over previous
import jax
import jax.numpy as jnp
from jax.experimental import pallas as pl

N = 10000
K = 16
D_IN = 256
D_OUT = 256
BLOCK = 1000


def _fused_kernel(src_ref, neigh_ref, wagg_ref, wself_ref, out_ref):
    neigh = neigh_ref[...]
    mean = jnp.sum(neigh, axis=1) * (1.0 / K)
    src = src_ref[...]
    h = jax.lax.dot_general(
        src, wself_ref[...], (((1,), (0,)), ((), ())),
        preferred_element_type=jnp.float32,
    )
    h += jax.lax.dot_general(
        mean, wagg_ref[...], (((1,), (0,)), ((), ())),
        preferred_element_type=jnp.float32,
    )
    out_ref[...] = jnp.maximum(h, 0.0)


def kernel(src_node_features, neighbor_node_features, W_agg, W_self):
    n = src_node_features.shape[0]
    grid = (n // BLOCK,)
    return pl.pallas_call(
        _fused_kernel,
        grid=grid,
        in_specs=[
            pl.BlockSpec((BLOCK, D_IN), lambda i: (i, 0)),
            pl.BlockSpec((BLOCK, K, D_IN), lambda i: (i, 0, 0)),
            pl.BlockSpec((D_IN, D_OUT), lambda i: (0, 0)),
            pl.BlockSpec((D_IN, D_OUT), lambda i: (0, 0)),
        ],
        out_specs=pl.BlockSpec((BLOCK, D_OUT), lambda i: (i, 0)),
        out_shape=jax.ShapeDtypeStruct((n, D_OUT), jnp.float32),
    )(src_node_features, neighbor_node_features, W_agg, W_self)
